# baseline (speedup 1.0000x reference)
import functools

import jax
import jax.numpy as jnp
from jax import lax
from jax.experimental import pallas as pl
from jax.experimental.pallas import tpu as pltpu
from jax.experimental.pallas import tpu_sc as plsc

N = 10000
E = 640000
D = 128
H = 64
G = 64
DE = 64

TPS = 16
CH = 16
NCH = 20
NBLK = NCH * CH
EPT = NBLK * 128
NP = 10112
RPW = NP // 16


def _deg_kernel_body(dstc_hbm, out_hbm, acc, ones_v, zbuf, idx_d):
    c = lax.axis_index("c")
    s = lax.axis_index("s")
    one16 = jnp.full((16,), 1.0, jnp.float32)
    zero16 = jnp.zeros((16,), jnp.float32)

    def _fill_ones(i, carry):
        ones_v[i] = one16
        return carry

    lax.fori_loop(0, 128, _fill_ones, 0)

    def _fill_zeros(i, carry):
        zbuf[i] = zero16
        return carry

    lax.fori_loop(0, 128, _fill_zeros, 0)

    base = s * RPW
    for t in range(4):
        pltpu.sync_copy(zbuf, acc.at[pl.ds(base + t * 128, 128)])
    pltpu.sync_copy(zbuf.at[pl.ds(0, RPW - 512)],
                    acc.at[pl.ds(base + 512, RPW - 512)])
    plsc.subcore_barrier()

    def _chunk(i, carry):
        ch = 2 * i + c
        pltpu.sync_copy(dstc_hbm.at[s, ch], idx_d)
        for k in range(CH):
            pltpu.sync_copy(ones_v, acc.at[idx_d.at[k]], add=True)
        return carry

    lax.fori_loop(0, NCH // 2, _chunk, 0)
    plsc.subcore_barrier()

    for t in range(4):
        pltpu.sync_copy(acc.at[pl.ds(base + t * 128, 128)], zbuf)
        pltpu.sync_copy(zbuf, out_hbm.at[c, pl.ds(base + t * 128, 128)])
    pltpu.sync_copy(acc.at[pl.ds(base + 512, RPW - 512)],
                    zbuf.at[pl.ds(0, RPW - 512)])
    pltpu.sync_copy(zbuf.at[pl.ds(0, RPW - 512)],
                    out_hbm.at[c, pl.ds(base + 512, RPW - 512)])


def _edge_kernel_body(u_hbm, srcc_hbm, dstc_hbm, out_hbm,
                      u_sh, acc, rows0, rows1, bounce,
                      idx_s0, idx_s1, idx_d0, idx_d1,
                      sem0, sem1, semi0, semi1):
    c = lax.axis_index("c")
    s = lax.axis_index("s")
    zero16 = jnp.zeros((16,), jnp.float32)

    def _fill_zeros(i, carry):
        for k in range(H // 16):
            rows0[i, pl.ds(k * 16, 16)] = zero16
        return carry

    lax.fori_loop(0, 128, _fill_zeros, 0)

    base = s * RPW
    for t in range(4):
        pltpu.sync_copy(rows0, acc.at[pl.ds(base + t * 128, 128)])
    pltpu.sync_copy(rows0.at[pl.ds(0, RPW - 512)],
                    acc.at[pl.ds(base + 512, RPW - 512)])
    def _take_half(i, carry):
        for k in range(H // 16):
            rows1[i, pl.ds(k * 16, 16)] = bounce[i, pl.ds(k * 16, 16)]
        return carry

    for t in range(9):
        pltpu.sync_copy(u_hbm.at[c, pl.ds(base + t * 64, 64)], bounce)
        lax.fori_loop(0, 64, _take_half, 0)
        pltpu.sync_copy(rows1.at[pl.ds(0, 64)],
                        u_sh.at[pl.ds(base + t * 64, 64)])
    pltpu.sync_copy(u_hbm.at[c, pl.ds(base + 576, RPW - 576)],
                    bounce.at[pl.ds(0, RPW - 576)])
    lax.fori_loop(0, RPW - 576, _take_half, 0)
    pltpu.sync_copy(rows1.at[pl.ds(0, RPW - 576)],
                    u_sh.at[pl.ds(base + 576, RPW - 576)])
    plsc.subcore_barrier()

    def _run_chunk(idx_s, idx_d):
        pltpu.async_copy(u_sh.at[idx_s.at[0]], rows0, sem0)
        pltpu.async_copy(u_sh.at[idx_s.at[1]], rows1, sem1)
        for k in range(CH):
            if k % 2 == 0:
                cur, csem = rows0, sem0
            else:
                cur, csem = rows1, sem1
            pltpu.make_async_copy(u_sh.at[idx_s.at[k]], cur, csem).wait()
            pltpu.sync_copy(cur, acc.at[idx_d.at[k]], add=True)
            if k + 2 < CH:
                pltpu.async_copy(u_sh.at[idx_s.at[k + 2]], cur, csem)

    pltpu.sync_copy(srcc_hbm.at[s, 0], idx_s0)
    pltpu.sync_copy(dstc_hbm.at[s, 0], idx_d0)

    def _pair(i, carry):
        pltpu.sync_copy(srcc_hbm.at[s, 2 * i + 1], idx_s1)
        pltpu.sync_copy(dstc_hbm.at[s, 2 * i + 1], idx_d1)
        _run_chunk(idx_s0, idx_d0)
        nxt = lax.rem(2 * i + 2, NCH)
        pltpu.sync_copy(srcc_hbm.at[s, nxt], idx_s0)
        pltpu.sync_copy(dstc_hbm.at[s, nxt], idx_d0)
        _run_chunk(idx_s1, idx_d1)
        return carry

    lax.fori_loop(0, NCH // 2, _pair, 0)
    plsc.subcore_barrier()

    def _put_half(i, carry):
        for k in range(H // 16):
            bounce[i, pl.ds(k * 16, 16)] = rows0[i, pl.ds(k * 16, 16)]
        return carry

    for t in range(9):
        pltpu.sync_copy(acc.at[pl.ds(base + t * 64, 64)],
                        rows0.at[pl.ds(0, 64)])
        lax.fori_loop(0, 64, _put_half, 0)
        pltpu.sync_copy(bounce, out_hbm.at[c, pl.ds(base + t * 64, 64)])
    pltpu.sync_copy(acc.at[pl.ds(base + 576, RPW - 576)],
                    rows0.at[pl.ds(0, RPW - 576)])
    lax.fori_loop(0, RPW - 576, _put_half, 0)
    pltpu.sync_copy(bounce.at[pl.ds(0, RPW - 576)],
                    out_hbm.at[c, pl.ds(base + 576, RPW - 576)])


@functools.cache
def _sc_kernels():
    mesh = plsc.VectorSubcoreMesh(core_axis_name="c", subcore_axis_name="s")
    deg_kernel = pl.kernel(
        _deg_kernel_body,
        out_type=jax.ShapeDtypeStruct((2, NP, 16), jnp.float32),
        mesh=mesh,
        scratch_types=[
            pltpu.VMEM_SHARED((NP, 16), jnp.float32),
            pltpu.VMEM((128, 16), jnp.float32),
            pltpu.VMEM((128, 16), jnp.float32),
            pltpu.VMEM((CH, 128), jnp.int32),
        ],
    )
    edge_kernel = pl.kernel(
        _edge_kernel_body,
        out_type=jax.ShapeDtypeStruct((2, NP, D), jnp.float32),
        mesh=mesh,
        scratch_types=[
            pltpu.VMEM_SHARED((NP, H), jnp.float32),
            pltpu.VMEM_SHARED((NP, H), jnp.float32),
            pltpu.VMEM((128, H), jnp.float32),
            pltpu.VMEM((128, H), jnp.float32),
            pltpu.VMEM((64, D), jnp.float32),
            pltpu.VMEM((CH, 128), jnp.int32),
            pltpu.VMEM((CH, 128), jnp.int32),
            pltpu.VMEM((CH, 128), jnp.int32),
            pltpu.VMEM((CH, 128), jnp.int32),
            pltpu.SemaphoreType.DMA,
            pltpu.SemaphoreType.DMA,
            pltpu.SemaphoreType.DMA,
            pltpu.SemaphoreType.DMA,
        ],
    )
    return deg_kernel, edge_kernel


def _dinv_from_degp(degp_ref):
    deg = degp_ref[0, :N, 0:1] + degp_ref[1, :N, 0:1] + 1.0
    return lax.rsqrt(deg)


def _split_u(uo, out_ref):
    out_ref[0, :N, :H] = uo[:, :H]
    out_ref[1, :N, :H] = uo[:, H:]


def _t0_body(x_ref, w_ref, degp_ref, out_ref):
    dinv = _dinv_from_degp(degp_ref)
    xw = jnp.dot(x_ref[...], w_ref[...], preferred_element_type=jnp.float32)
    _split_u(xw * dinv, out_ref)


_t0 = pl.pallas_call(
    _t0_body, out_shape=jax.ShapeDtypeStruct((2, NP, D), jnp.float32))


def _post(sp_ref, u_ref, degp_ref, b_ref, g_ref, beta_ref):
    dinv = _dinv_from_degp(degp_ref)
    su = jnp.concatenate(
        [sp_ref[0, :N, :H] + u_ref[0, :N, :H],
         sp_ref[1, :N, :H] + u_ref[1, :N, :H]], axis=1)
    t = su * dinv + b_ref[...]
    mean = jnp.mean(t, axis=0, keepdims=True)
    var = jnp.mean(jnp.square(t - mean), axis=0, keepdims=True)
    h = (t - mean) * lax.rsqrt(var + 1e-5) * g_ref[...] + beta_ref[...]
    return jnp.maximum(h, 0.0), dinv


def _mid_body(sp_ref, u_ref, degp_ref, b_ref, g_ref, beta_ref, w_ref, out_ref):
    h, dinv = _post(sp_ref, u_ref, degp_ref, b_ref, g_ref, beta_ref)
    uo = jnp.dot(h, w_ref[...], preferred_element_type=jnp.float32) * dinv
    _split_u(uo, out_ref)


_mid = pl.pallas_call(
    _mid_body, out_shape=jax.ShapeDtypeStruct((2, NP, D), jnp.float32))


def _fin_body(sp_ref, u_ref, degp_ref, b_ref, g_ref, beta_ref,
              batch_ref, we_ref, be_ref, out_ref):
    h, _ = _post(sp_ref, u_ref, degp_ref, b_ref, g_ref, beta_ref)
    gid = lax.broadcasted_iota(jnp.int32, (1, G), 1)
    mask = (batch_ref[...] == gid).astype(jnp.float32)
    sums = lax.dot_general(mask, h, (((0,), (0,)), ((), ())),
                           preferred_element_type=jnp.float32)
    ones_col = jnp.ones((N, 1), jnp.float32)
    cnts = lax.dot_general(mask, ones_col, (((0,), (0,)), ((), ())),
                           preferred_element_type=jnp.float32)
    pooled = sums / jnp.maximum(cnts, 1.0)
    out_ref[...] = jnp.dot(
        pooled, we_ref[...], preferred_element_type=jnp.float32) + be_ref[...]


_fin = pl.pallas_call(
    _fin_body, out_shape=jax.ShapeDtypeStruct((G, DE), jnp.float32))


def kernel(x, edge_index, batch, W0, b0, W1, b1, W2, b2,
           g0, be0, g1, be1, g2, be2, We, be):
    src = edge_index[0]
    dst = edge_index[1]
    ppt = EPT - E // TPS
    dummy = N + (jnp.arange(ppt, dtype=jnp.int32) % (NP - N))
    srcp = jnp.concatenate(
        [src.reshape(TPS, E // TPS),
         jnp.zeros((TPS, ppt), jnp.int32)],
        axis=1).reshape(TPS, NCH, CH, 128)
    dstp = jnp.concatenate(
        [dst.reshape(TPS, E // TPS),
         jnp.broadcast_to(dummy, (TPS, ppt))],
        axis=1).reshape(TPS, NCH, CH, 128)

    deg_kernel, edge_kernel = _sc_kernels()
    degp = deg_kernel(dstp)
    b0r, g0r, be0r = b0.reshape(1, D), g0.reshape(1, D), be0.reshape(1, D)
    b1r, g1r, be1r = b1.reshape(1, D), g1.reshape(1, D), be1.reshape(1, D)
    b2r, g2r, be2r = b2.reshape(1, D), g2.reshape(1, D), be2.reshape(1, D)

    u = _t0(x, W0, degp)
    sp = edge_kernel(u, srcp, dstp)
    u = _mid(sp, u, degp, b0r, g0r, be0r, W1)
    sp = edge_kernel(u, srcp, dstp)
    u = _mid(sp, u, degp, b1r, g1r, be1r, W2)
    sp = edge_kernel(u, srcp, dstp)
    return _fin(sp, u, degp, b2r, g2r, be2r,
                batch.reshape(N, 1), We, be.reshape(1, DE))


# speedup vs baseline: 1.0790x; 1.0790x over previous
import functools

import jax
import jax.numpy as jnp
from jax import lax
from jax.experimental import pallas as pl
from jax.experimental.pallas import tpu as pltpu
from jax.experimental.pallas import tpu_sc as plsc

N = 10000
E = 640000
D = 128
H = 64
G = 64
DE = 64

TPS = 16
CH = 32
NCH = 10
NBLK = NCH * CH
EPT = NBLK * 128
NP = 10112
RPW = NP // 16


def _deg_kernel_body(dstc_hbm, out_hbm, acc, ones_v, zbuf, idx_d):
    c = lax.axis_index("c")
    s = lax.axis_index("s")
    one16 = jnp.full((16,), 1.0, jnp.float32)
    zero16 = jnp.zeros((16,), jnp.float32)

    def _fill_ones(i, carry):
        ones_v[i] = one16
        return carry

    lax.fori_loop(0, 128, _fill_ones, 0)

    def _fill_zeros(i, carry):
        zbuf[i] = zero16
        return carry

    lax.fori_loop(0, 128, _fill_zeros, 0)

    base = s * RPW
    for t in range(4):
        pltpu.sync_copy(zbuf, acc.at[pl.ds(base + t * 128, 128)])
    pltpu.sync_copy(zbuf.at[pl.ds(0, RPW - 512)],
                    acc.at[pl.ds(base + 512, RPW - 512)])
    plsc.subcore_barrier()

    def _chunk(i, carry):
        ch = 2 * i + c
        pltpu.sync_copy(dstc_hbm.at[s, ch], idx_d)
        for k in range(CH):
            pltpu.sync_copy(ones_v, acc.at[idx_d.at[k]], add=True)
        return carry

    lax.fori_loop(0, NCH // 2, _chunk, 0)
    plsc.subcore_barrier()

    for t in range(4):
        pltpu.sync_copy(acc.at[pl.ds(base + t * 128, 128)], zbuf)
        pltpu.sync_copy(zbuf, out_hbm.at[c, pl.ds(base + t * 128, 128)])
    pltpu.sync_copy(acc.at[pl.ds(base + 512, RPW - 512)],
                    zbuf.at[pl.ds(0, RPW - 512)])
    pltpu.sync_copy(zbuf.at[pl.ds(0, RPW - 512)],
                    out_hbm.at[c, pl.ds(base + 512, RPW - 512)])


def _edge_kernel_body(u_hbm, srcc_hbm, dstc_hbm, out_hbm,
                      u_sh, acc, rows0, rows1, bounce,
                      idx_s, idx_d, sem0, sem1):
    c = lax.axis_index("c")
    s = lax.axis_index("s")
    zero16 = jnp.zeros((16,), jnp.float32)

    def _fill_zeros(i, carry):
        for k in range(H // 16):
            rows0[i, pl.ds(k * 16, 16)] = zero16
        return carry

    lax.fori_loop(0, 128, _fill_zeros, 0)

    base = s * RPW
    for t in range(4):
        pltpu.sync_copy(rows0, acc.at[pl.ds(base + t * 128, 128)])
    pltpu.sync_copy(rows0.at[pl.ds(0, RPW - 512)],
                    acc.at[pl.ds(base + 512, RPW - 512)])
    def _take_half(i, carry):
        for k in range(H // 16):
            rows1[i, pl.ds(k * 16, 16)] = bounce[i, pl.ds(k * 16, 16)]
        return carry

    for t in range(9):
        pltpu.sync_copy(u_hbm.at[c, pl.ds(base + t * 64, 64)], bounce)
        lax.fori_loop(0, 64, _take_half, 0)
        pltpu.sync_copy(rows1.at[pl.ds(0, 64)],
                        u_sh.at[pl.ds(base + t * 64, 64)])
    pltpu.sync_copy(u_hbm.at[c, pl.ds(base + 576, RPW - 576)],
                    bounce.at[pl.ds(0, RPW - 576)])
    lax.fori_loop(0, RPW - 576, _take_half, 0)
    pltpu.sync_copy(rows1.at[pl.ds(0, RPW - 576)],
                    u_sh.at[pl.ds(base + 576, RPW - 576)])
    plsc.subcore_barrier()

    def _chunk(ch, carry):
        pltpu.sync_copy(srcc_hbm.at[s, ch], idx_s)
        pltpu.sync_copy(dstc_hbm.at[s, ch], idx_d)
        pltpu.async_copy(u_sh.at[idx_s.at[0]], rows0, sem0)
        pltpu.async_copy(u_sh.at[idx_s.at[1]], rows1, sem1)
        for k in range(CH):
            if k % 2 == 0:
                cur, csem = rows0, sem0
            else:
                cur, csem = rows1, sem1
            pltpu.make_async_copy(u_sh.at[idx_s.at[k]], cur, csem).wait()
            pltpu.sync_copy(cur, acc.at[idx_d.at[k]], add=True)
            if k + 2 < CH:
                pltpu.async_copy(u_sh.at[idx_s.at[k + 2]], cur, csem)
        return carry

    lax.fori_loop(0, NCH, _chunk, 0)
    plsc.subcore_barrier()

    def _put_half(i, carry):
        for k in range(H // 16):
            bounce[i, pl.ds(k * 16, 16)] = rows0[i, pl.ds(k * 16, 16)]
        return carry

    for t in range(9):
        pltpu.sync_copy(acc.at[pl.ds(base + t * 64, 64)],
                        rows0.at[pl.ds(0, 64)])
        lax.fori_loop(0, 64, _put_half, 0)
        pltpu.sync_copy(bounce, out_hbm.at[c, pl.ds(base + t * 64, 64)])
    pltpu.sync_copy(acc.at[pl.ds(base + 576, RPW - 576)],
                    rows0.at[pl.ds(0, RPW - 576)])
    lax.fori_loop(0, RPW - 576, _put_half, 0)
    pltpu.sync_copy(bounce.at[pl.ds(0, RPW - 576)],
                    out_hbm.at[c, pl.ds(base + 576, RPW - 576)])


@functools.cache
def _sc_kernels():
    mesh = plsc.VectorSubcoreMesh(core_axis_name="c", subcore_axis_name="s")
    deg_kernel = pl.kernel(
        _deg_kernel_body,
        out_type=jax.ShapeDtypeStruct((2, NP, 16), jnp.float32),
        mesh=mesh,
        scratch_types=[
            pltpu.VMEM_SHARED((NP, 16), jnp.float32),
            pltpu.VMEM((128, 16), jnp.float32),
            pltpu.VMEM((128, 16), jnp.float32),
            pltpu.VMEM((CH, 128), jnp.int32),
        ],
    )
    edge_kernel = pl.kernel(
        _edge_kernel_body,
        out_type=jax.ShapeDtypeStruct((2, NP, D), jnp.float32),
        mesh=mesh,
        scratch_types=[
            pltpu.VMEM_SHARED((NP, H), jnp.float32),
            pltpu.VMEM_SHARED((NP, H), jnp.float32),
            pltpu.VMEM((128, H), jnp.float32),
            pltpu.VMEM((128, H), jnp.float32),
            pltpu.VMEM((64, D), jnp.float32),
            pltpu.VMEM((CH, 128), jnp.int32),
            pltpu.VMEM((CH, 128), jnp.int32),
            pltpu.SemaphoreType.DMA,
            pltpu.SemaphoreType.DMA,
        ],
    )
    return deg_kernel, edge_kernel


def _dinv_from_degp(degp_ref):
    deg = degp_ref[0, :N, 0:1] + degp_ref[1, :N, 0:1] + 1.0
    return lax.rsqrt(deg)


def _split_u(uo, out_ref):
    out_ref[0, :N, :H] = uo[:, :H]
    out_ref[1, :N, :H] = uo[:, H:]


def _t0_body(x_ref, w_ref, degp_ref, out_ref):
    dinv = _dinv_from_degp(degp_ref)
    xw = jnp.dot(x_ref[...], w_ref[...], preferred_element_type=jnp.float32)
    _split_u(xw * dinv, out_ref)


_t0 = pl.pallas_call(
    _t0_body, out_shape=jax.ShapeDtypeStruct((2, NP, D), jnp.float32))


def _post(sp_ref, u_ref, degp_ref, b_ref, g_ref, beta_ref):
    dinv = _dinv_from_degp(degp_ref)
    su = jnp.concatenate(
        [sp_ref[0, :N, :H] + u_ref[0, :N, :H],
         sp_ref[1, :N, :H] + u_ref[1, :N, :H]], axis=1)
    t = su * dinv + b_ref[...]
    mean = jnp.mean(t, axis=0, keepdims=True)
    var = jnp.mean(jnp.square(t - mean), axis=0, keepdims=True)
    h = (t - mean) * lax.rsqrt(var + 1e-5) * g_ref[...] + beta_ref[...]
    return jnp.maximum(h, 0.0), dinv


def _mid_body(sp_ref, u_ref, degp_ref, b_ref, g_ref, beta_ref, w_ref, out_ref):
    h, dinv = _post(sp_ref, u_ref, degp_ref, b_ref, g_ref, beta_ref)
    uo = jnp.dot(h, w_ref[...], preferred_element_type=jnp.float32) * dinv
    _split_u(uo, out_ref)


_mid = pl.pallas_call(
    _mid_body, out_shape=jax.ShapeDtypeStruct((2, NP, D), jnp.float32))


def _fin_body(sp_ref, u_ref, degp_ref, b_ref, g_ref, beta_ref,
              batch_ref, we_ref, be_ref, out_ref):
    h, _ = _post(sp_ref, u_ref, degp_ref, b_ref, g_ref, beta_ref)
    gid = lax.broadcasted_iota(jnp.int32, (1, G), 1)
    mask = (batch_ref[...] == gid).astype(jnp.float32)
    sums = lax.dot_general(mask, h, (((0,), (0,)), ((), ())),
                           preferred_element_type=jnp.float32)
    ones_col = jnp.ones((N, 1), jnp.float32)
    cnts = lax.dot_general(mask, ones_col, (((0,), (0,)), ((), ())),
                           preferred_element_type=jnp.float32)
    pooled = sums / jnp.maximum(cnts, 1.0)
    out_ref[...] = jnp.dot(
        pooled, we_ref[...], preferred_element_type=jnp.float32) + be_ref[...]


_fin = pl.pallas_call(
    _fin_body, out_shape=jax.ShapeDtypeStruct((G, DE), jnp.float32))


def kernel(x, edge_index, batch, W0, b0, W1, b1, W2, b2,
           g0, be0, g1, be1, g2, be2, We, be):
    src = edge_index[0]
    dst = edge_index[1]
    ppt = EPT - E // TPS
    dummy = N + (jnp.arange(ppt, dtype=jnp.int32) % (NP - N))
    srcp = jnp.concatenate(
        [src.reshape(TPS, E // TPS),
         jnp.zeros((TPS, ppt), jnp.int32)],
        axis=1).reshape(TPS, NCH, CH, 128)
    dstp = jnp.concatenate(
        [dst.reshape(TPS, E // TPS),
         jnp.broadcast_to(dummy, (TPS, ppt))],
        axis=1).reshape(TPS, NCH, CH, 128)

    deg_kernel, edge_kernel = _sc_kernels()
    degp = deg_kernel(dstp)
    b0r, g0r, be0r = b0.reshape(1, D), g0.reshape(1, D), be0.reshape(1, D)
    b1r, g1r, be1r = b1.reshape(1, D), g1.reshape(1, D), be1.reshape(1, D)
    b2r, g2r, be2r = b2.reshape(1, D), g2.reshape(1, D), be2.reshape(1, D)

    u = _t0(x, W0, degp)
    sp = edge_kernel(u, srcp, dstp)
    u = _mid(sp, u, degp, b0r, g0r, be0r, W1)
    sp = edge_kernel(u, srcp, dstp)
    u = _mid(sp, u, degp, b1r, g1r, be1r, W2)
    sp = edge_kernel(u, srcp, dstp)
    return _fin(sp, u, degp, b2r, g2r, be2r,
                batch.reshape(N, 1), We, be.reshape(1, DE))
